# baseline (speedup 1.0000x reference)
import functools

import jax
import jax.numpy as jnp
from jax import lax
from jax.experimental import pallas as pl
from jax.experimental.pallas import tpu as pltpu
from jax.experimental.pallas import tpu_sc as plsc

NC = 2
NS = 16
L = 16
NW = NC * NS

N = 10000
E = 10000
D = 128
R_PAD = 10240
TRASH = 10000
CHUNK = 128
K = 80
NNZ_PAD = NW * K * CHUNK
RPT = R_PAD // NS
RPW = R_PAD // NW
NB = 64


def _mesh():
  return plsc.VectorSubcoreMesh(
      core_axis_name="c", subcore_axis_name="s", num_cores=NC, num_subcores=NS
  )


def _scatter_body(with_counts, *refs):
  if with_counts:
    (src_hbm, idx_hbm, out_sums, out_cnt,
     idx_v, rows2_v, cnt_v,
     acc_sh, sem_g) = refs
  else:
    (src_hbm, idx_hbm, out_sums,
     idx_v, rows2_v,
     acc_sh, sem_g) = refs
    cnt_v = None
  rows_v = rows2_v

  c = lax.axis_index("c")
  s = lax.axis_index("s")
  wid = c * NS + s
  base = s * RPT

  zv = jnp.zeros((L,), jnp.float32)

  def zero_row(i, carry):
    for j in range(D // L):
      rows_v[i, pl.ds(j * L, L)] = zv
    return carry

  lax.fori_loop(0, CHUNK, zero_row, 0)

  if with_counts:
    def zero_cnt(i, carry):
      cnt_v[pl.ds(i * L, L)] = zv
      return carry

    lax.fori_loop(0, R_PAD // L, zero_cnt, 0)

  for k in range(RPT // CHUNK):
    pltpu.sync_copy(rows_v, acc_sh.at[pl.ds(base + k * CHUNK, CHUNK)])

  pltpu.sync_copy(idx_hbm.at[wid], idx_v)

  plsc.subcore_barrier()

  ones = jnp.ones((L,), jnp.float32)
  buf = rows2_v

  def step(jj, carry):
    pltpu.async_copy(src_hbm.at[idx_v.at[jj, 0]], buf, sem_g).wait()
    pltpu.sync_copy(buf, acc_sh.at[idx_v.at[jj, 1]], add=True)
    if with_counts:
      for t in range(CHUNK // L):
        idx = idx_v[jj, 1, pl.ds(t * L, L)]
        plsc.addupdate_scatter(cnt_v, [idx], ones)
    return carry

  lax.fori_loop(0, K, step, 0)

  if with_counts:
    pltpu.sync_copy(cnt_v, out_cnt.at[pl.ds(wid * R_PAD, R_PAD)])

  plsc.subcore_barrier()

  for k in range(RPT // CHUNK):
    r0 = base + k * CHUNK
    pltpu.sync_copy(acc_sh.at[pl.ds(r0, CHUNK)], rows_v)
    pltpu.sync_copy(rows_v, out_sums.at[c, pl.ds(r0, CHUNK)])


def _make_scatter(with_counts):
  outs = [jax.ShapeDtypeStruct((NC, R_PAD, D), jnp.float32)]
  scratch = [
      pltpu.VMEM((K, 2, CHUNK), jnp.int32),
      pltpu.VMEM((CHUNK, D), jnp.float32),
  ]
  if with_counts:
    outs.append(jax.ShapeDtypeStruct((NW * R_PAD,), jnp.float32))
    scratch.append(pltpu.VMEM((R_PAD,), jnp.float32))
  scratch.append(pltpu.VMEM_SHARED((R_PAD, D), jnp.float32))
  scratch.append(pltpu.SemaphoreType.DMA)
  return pl.kernel(
      functools.partial(_scatter_body, with_counts),
      out_type=tuple(outs) if with_counts else outs[0],
      mesh=_mesh(),
      scratch_types=scratch,
      compiler_params=pltpu.CompilerParams(needs_layout_passes=False),
  )


def _tc_norm_body(s0_ref, s1_ref, cnt_ref, o_ref):
  c = jnp.sum(cnt_ref[...], axis=0)
  scale = 1.0 / jnp.maximum(c, 1.0)
  o_ref[...] = (s0_ref[...] + s1_ref[...]) * scale[:, None]


def _tc_norm(s0, s1, cnt2):
  BR = 1024
  return pl.pallas_call(
      _tc_norm_body,
      grid=(R_PAD // BR,),
      in_specs=[
          pl.BlockSpec((BR, D), lambda i: (i, 0)),
          pl.BlockSpec((BR, D), lambda i: (i, 0)),
          pl.BlockSpec((NW, BR), lambda i: (0, i)),
      ],
      out_specs=pl.BlockSpec((BR, D), lambda i: (i, 0)),
      out_shape=jax.ShapeDtypeStruct((R_PAD, D), jnp.float32),
  )(s0, s1, cnt2)


def _mm_body(eps_ref, x_ref, v0_ref, v1_ref, w_ref, o_ref):
  scale = 1.0 + eps_ref[0]
  acc = scale * x_ref[...] + v0_ref[...] + v1_ref[...]
  o_ref[...] = lax.dot_general(
      acc, w_ref[...], (((1,), (1,)), ((), ())),
      preferred_element_type=jnp.float32,
  )


def _matmul(eps, Xp, v0, v1, W):
  M = Xp.shape[0]
  BM = 1280
  return pl.pallas_call(
      _mm_body,
      grid=(M // BM,),
      in_specs=[
          pl.BlockSpec(memory_space=pltpu.SMEM),
          pl.BlockSpec((BM, D), lambda i: (i, 0)),
          pl.BlockSpec((BM, D), lambda i: (i, 0)),
          pl.BlockSpec((BM, D), lambda i: (i, 0)),
          pl.BlockSpec((D, D), lambda i: (0, 0)),
      ],
      out_specs=pl.BlockSpec((BM, D), lambda i: (i, 0)),
      out_shape=jax.ShapeDtypeStruct((M, D), jnp.float32),
  )(eps, Xp, v0, v1, W)


def kernel(X, vertex, edges, W, eps):
  Xp = jnp.pad(X, ((0, R_PAD - N), (0, 0)))
  npad = NNZ_PAD - vertex.shape[0]
  fill = jnp.full((npad,), TRASH, jnp.int32)
  v3 = jnp.concatenate([vertex.astype(jnp.int32), fill]).reshape(NW, K, CHUNK)
  e3 = jnp.concatenate([edges.astype(jnp.int32), fill]).reshape(NW, K, CHUNK)

  idx_ve = jnp.stack([v3, e3], axis=2)
  idx_ev = jnp.stack([e3, v3], axis=2)
  sums, cnts = _make_scatter(True)(Xp, idx_ve)
  Xe = _tc_norm(sums[0], sums[1], cnts.reshape(NW, R_PAD))
  xv = _make_scatter(False)(Xe, idx_ev)
  out = _matmul(eps, Xp, xv[0], xv[1], W)
  return out[:N]


# speedup vs baseline: 2.7013x; 2.7013x over previous
import functools

import jax
import jax.numpy as jnp
from jax import lax
from jax.experimental import pallas as pl
from jax.experimental.pallas import tpu as pltpu
from jax.experimental.pallas import tpu_sc as plsc

NC = 2
NS = 16
L = 16
NW = NC * NS

N = 10000
E = 10000
D = 128
R_PAD = 10240
TRASH = 10000
CHUNK = 128
K = 80
NNZ_PAD = NW * K * CHUNK
RPT = R_PAD // NS
RPW = R_PAD // NW
NB = 64


def _mesh():
  return plsc.VectorSubcoreMesh(
      core_axis_name="c", subcore_axis_name="s", num_cores=NC, num_subcores=NS
  )


def _scatter_body(with_counts, *refs):
  if with_counts:
    (src_hbm, idx_hbm, out_sums, out_cnt,
     idx_v, rows2_v, cnt_v,
     acc_sh, sem_g) = refs
  else:
    (src_hbm, idx_hbm, out_sums,
     idx_v, rows2_v,
     acc_sh, sem_g) = refs
    cnt_v = None
  rows_v = rows2_v

  c = lax.axis_index("c")
  s = lax.axis_index("s")
  wid = c * NS + s
  base = s * RPT

  zv = jnp.zeros((L,), jnp.float32)

  def zero_row(i, carry):
    for j in range(D // L):
      rows_v[i, pl.ds(j * L, L)] = zv
    return carry

  lax.fori_loop(0, CHUNK, zero_row, 0)

  if with_counts:
    def zero_cnt(i, carry):
      cnt_v[pl.ds(i * L, L)] = zv
      return carry

    lax.fori_loop(0, R_PAD // L, zero_cnt, 0)

  for k in range(RPT // CHUNK):
    pltpu.sync_copy(rows_v, acc_sh.at[pl.ds(base + k * CHUNK, CHUNK)])

  pltpu.sync_copy(idx_hbm.at[wid], idx_v)

  plsc.subcore_barrier()

  ones = jnp.ones((L,), jnp.float32)
  buf = rows2_v

  def step(jj, carry):
    pltpu.async_copy(src_hbm.at[idx_v.at[jj, 0]], buf, sem_g).wait()
    pltpu.sync_copy(buf, acc_sh.at[idx_v.at[jj, 1]], add=True)
    if with_counts:
      for t in range(CHUNK // L):
        idx = idx_v[jj, 1, pl.ds(t * L, L)]
        plsc.addupdate_scatter(cnt_v, [idx], ones)
    return carry

  lax.fori_loop(0, K, step, 0)

  if with_counts:
    pltpu.sync_copy(cnt_v, out_cnt.at[pl.ds(wid * R_PAD, R_PAD)])

  plsc.subcore_barrier()

  for k in range(RPT // CHUNK):
    r0 = base + k * CHUNK
    pltpu.sync_copy(acc_sh.at[pl.ds(r0, CHUNK)], rows_v)
    pltpu.sync_copy(rows_v, out_sums.at[c, pl.ds(r0, CHUNK)])


def _make_scatter(with_counts):
  outs = [jax.ShapeDtypeStruct((NC, R_PAD, D), jnp.float32)]
  scratch = [
      pltpu.VMEM((K, 2, CHUNK), jnp.int32),
      pltpu.VMEM((CHUNK, D), jnp.float32),
  ]
  if with_counts:
    outs.append(jax.ShapeDtypeStruct((NW * R_PAD,), jnp.float32))
    scratch.append(pltpu.VMEM((R_PAD,), jnp.float32))
  scratch.append(pltpu.VMEM_SHARED((R_PAD, D), jnp.float32))
  scratch.append(pltpu.SemaphoreType.DMA)
  return pl.kernel(
      functools.partial(_scatter_body, with_counts),
      out_type=tuple(outs) if with_counts else outs[0],
      mesh=_mesh(),
      scratch_types=scratch,
      compiler_params=pltpu.CompilerParams(needs_layout_passes=False),
  )


def _tc_norm_body(s0_ref, s1_ref, cnt_ref, o_ref):
  c = jnp.sum(cnt_ref[...], axis=0)
  scale = 1.0 / jnp.maximum(c, 1.0)
  o_ref[...] = (s0_ref[...] + s1_ref[...]) * scale[:, None]


def _tc_norm(s0, s1, cnt2):
  BR = 1024
  return pl.pallas_call(
      _tc_norm_body,
      grid=(R_PAD // BR,),
      in_specs=[
          pl.BlockSpec((BR, D), lambda i: (i, 0)),
          pl.BlockSpec((BR, D), lambda i: (i, 0)),
          pl.BlockSpec((NW, BR), lambda i: (0, i)),
      ],
      out_specs=pl.BlockSpec((BR, D), lambda i: (i, 0)),
      out_shape=jax.ShapeDtypeStruct((R_PAD, D), jnp.float32),
  )(s0, s1, cnt2)


def _mm_body(eps_ref, x_ref, v0_ref, v1_ref, w_ref, o_ref):
  scale = 1.0 + eps_ref[0]
  acc = scale * x_ref[...] + v0_ref[...] + v1_ref[...]
  o_ref[...] = lax.dot_general(
      acc, w_ref[...], (((1,), (1,)), ((), ())),
      preferred_element_type=jnp.float32,
  )


def _matmul(eps, Xp, v0, v1, W):
  M = Xp.shape[0]
  BM = 1280
  return pl.pallas_call(
      _mm_body,
      grid=(M // BM,),
      in_specs=[
          pl.BlockSpec(memory_space=pltpu.SMEM),
          pl.BlockSpec((BM, D), lambda i: (i, 0)),
          pl.BlockSpec((BM, D), lambda i: (i, 0)),
          pl.BlockSpec((BM, D), lambda i: (i, 0)),
          pl.BlockSpec((D, D), lambda i: (0, 0)),
      ],
      out_specs=pl.BlockSpec((BM, D), lambda i: (i, 0)),
      out_shape=jax.ShapeDtypeStruct((M, D), jnp.float32),
  )(eps, Xp, v0, v1, W)


def kernel(X, vertex, edges, W, eps):
  Xp = jnp.pad(X, ((0, R_PAD - N), (0, 0)))
  npad = NNZ_PAD - vertex.shape[0]
  fill = TRASH + (jnp.arange(npad, dtype=jnp.int32) % (R_PAD - N))
  v3 = jnp.concatenate([vertex.astype(jnp.int32), fill]).reshape(NW, K, CHUNK)
  e3 = jnp.concatenate([edges.astype(jnp.int32), fill]).reshape(NW, K, CHUNK)

  idx_ve = jnp.stack([v3, e3], axis=2)
  idx_ev = jnp.stack([e3, v3], axis=2)
  sums, cnts = _make_scatter(True)(Xp, idx_ve)
  Xe = _tc_norm(sums[0], sums[1], cnts.reshape(NW, R_PAD))
  xv = _make_scatter(False)(Xe, idx_ev)
  out = _matmul(eps, Xp, xv[0], xv[1], W)
  return out[:N]


# speedup vs baseline: 4.0034x; 1.4820x over previous
import functools

import jax
import jax.numpy as jnp
from jax import lax
from jax.experimental import pallas as pl
from jax.experimental.pallas import tpu as pltpu
from jax.experimental.pallas import tpu_sc as plsc

NC = 2
NS = 16
L = 16
NW = NC * NS

N = 10000
E = 10000
D = 128
R_PAD = 10240
TRASH = 10000
CHUNK = 128
K = 80
NNZ_PAD = NW * K * CHUNK
RPT = R_PAD // NS
RPW = R_PAD // NW
NB = 64


def _mesh():
  return plsc.VectorSubcoreMesh(
      core_axis_name="c", subcore_axis_name="s", num_cores=NC, num_subcores=NS
  )


def _scatter_body(with_counts, *refs):
  if with_counts:
    (src_hbm, idx_hbm, out_sums, out_cnt,
     idx_v, rows2_v, cnt_v,
     acc_sh, sem_g0, sem_g1) = refs
  else:
    (src_hbm, idx_hbm, out_sums,
     idx_v, rows2_v,
     acc_sh, sem_g0, sem_g1) = refs
    cnt_v = None
  rows_v = rows2_v.at[0]

  c = lax.axis_index("c")
  s = lax.axis_index("s")
  wid = c * NS + s
  base = s * RPT

  zv = jnp.zeros((L,), jnp.float32)

  def zero_row(i, carry):
    for j in range(D // L):
      rows_v[i, pl.ds(j * L, L)] = zv
    return carry

  lax.fori_loop(0, CHUNK, zero_row, 0)

  if with_counts:
    def zero_cnt(i, carry):
      cnt_v[pl.ds(i * L, L)] = zv
      return carry

    lax.fori_loop(0, R_PAD // L, zero_cnt, 0)

  for k in range(RPT // CHUNK):
    pltpu.sync_copy(rows_v, acc_sh.at[pl.ds(base + k * CHUNK, CHUNK)])

  pltpu.sync_copy(idx_hbm.at[wid, 0], idx_v.at[0])

  plsc.subcore_barrier()

  ones = jnp.ones((L,), jnp.float32)

  pltpu.async_copy(src_hbm.at[idx_v.at[0, 0, 0]], rows2_v.at[0], sem_g0)
  pltpu.async_copy(src_hbm.at[idx_v.at[0, 1, 0]], rows2_v.at[1], sem_g1)

  def do_counts(blk, row):
    for t in range(CHUNK // L):
      idx = idx_v[blk, row, 1, pl.ds(t * L, L)]
      plsc.addupdate_scatter(cnt_v, [idx], ones)

  def half(j, buf, sem):
    blk = jnp.bitwise_and(lax.shift_right_logical(j, 3), 1)
    row = jnp.bitwise_and(j, 7)
    pltpu.make_async_copy(src_hbm.at[idx_v.at[blk, row, 0]], buf, sem).wait()
    pltpu.sync_copy(buf, acc_sh.at[idx_v.at[blk, row, 1]], add=True)

    @pl.when(j + 2 < K)
    def _():
      blk2 = jnp.bitwise_and(lax.shift_right_logical(j + 2, 3), 1)
      row2 = jnp.bitwise_and(j + 2, 7)
      pltpu.async_copy(src_hbm.at[idx_v.at[blk2, row2, 0]], buf, sem)

    if with_counts:
      do_counts(blk, row)

  def pair(i, carry):
    j = 2 * i

    @pl.when(jnp.logical_and(jnp.bitwise_and(j, 7) == 6, j + 2 < K))
    def _():
      nxt = lax.shift_right_logical(j + 2, 3)
      pltpu.sync_copy(idx_hbm.at[wid, nxt],
                      idx_v.at[jnp.bitwise_and(nxt, 1)])

    half(j, rows2_v.at[0], sem_g0)
    half(j + 1, rows2_v.at[1], sem_g1)
    return carry

  lax.fori_loop(0, K // 2, pair, 0)

  if with_counts:
    pltpu.sync_copy(cnt_v, out_cnt.at[pl.ds(wid * R_PAD, R_PAD)])

  plsc.subcore_barrier()

  for k in range(RPT // CHUNK):
    r0 = base + k * CHUNK
    pltpu.sync_copy(acc_sh.at[pl.ds(r0, CHUNK)], rows_v)
    pltpu.sync_copy(rows_v, out_sums.at[c, pl.ds(r0, CHUNK)])


def _make_scatter(with_counts):
  outs = [jax.ShapeDtypeStruct((NC, R_PAD, D), jnp.float32)]
  scratch = [
      pltpu.VMEM((2, 8, 2, CHUNK), jnp.int32),
      pltpu.VMEM((2, CHUNK, D), jnp.float32),
  ]
  if with_counts:
    outs.append(jax.ShapeDtypeStruct((NW * R_PAD,), jnp.float32))
    scratch.append(pltpu.VMEM((R_PAD,), jnp.float32))
  scratch.append(pltpu.VMEM_SHARED((R_PAD, D), jnp.float32))
  scratch += [pltpu.SemaphoreType.DMA] * 2
  return pl.kernel(
      functools.partial(_scatter_body, with_counts),
      out_type=tuple(outs) if with_counts else outs[0],
      mesh=_mesh(),
      scratch_types=scratch,
      compiler_params=pltpu.CompilerParams(needs_layout_passes=False),
  )


def _tc_norm_body(s0_ref, s1_ref, cnt_ref, o_ref):
  c = jnp.sum(cnt_ref[...], axis=0)
  scale = 1.0 / jnp.maximum(c, 1.0)
  o_ref[...] = (s0_ref[...] + s1_ref[...]) * scale[:, None]


def _tc_norm(s0, s1, cnt2):
  BR = 1024
  return pl.pallas_call(
      _tc_norm_body,
      grid=(R_PAD // BR,),
      in_specs=[
          pl.BlockSpec((BR, D), lambda i: (i, 0)),
          pl.BlockSpec((BR, D), lambda i: (i, 0)),
          pl.BlockSpec((NW, BR), lambda i: (0, i)),
      ],
      out_specs=pl.BlockSpec((BR, D), lambda i: (i, 0)),
      out_shape=jax.ShapeDtypeStruct((R_PAD, D), jnp.float32),
  )(s0, s1, cnt2)


def _mm_body(eps_ref, x_ref, v0_ref, v1_ref, w_ref, o_ref):
  scale = 1.0 + eps_ref[0]
  acc = scale * x_ref[...] + v0_ref[...] + v1_ref[...]
  o_ref[...] = lax.dot_general(
      acc, w_ref[...], (((1,), (1,)), ((), ())),
      preferred_element_type=jnp.float32,
  )


def _matmul(eps, Xp, v0, v1, W):
  M = Xp.shape[0]
  BM = 1280
  return pl.pallas_call(
      _mm_body,
      grid=(M // BM,),
      in_specs=[
          pl.BlockSpec(memory_space=pltpu.SMEM),
          pl.BlockSpec((BM, D), lambda i: (i, 0)),
          pl.BlockSpec((BM, D), lambda i: (i, 0)),
          pl.BlockSpec((BM, D), lambda i: (i, 0)),
          pl.BlockSpec((D, D), lambda i: (0, 0)),
      ],
      out_specs=pl.BlockSpec((BM, D), lambda i: (i, 0)),
      out_shape=jax.ShapeDtypeStruct((M, D), jnp.float32),
  )(eps, Xp, v0, v1, W)


def kernel(X, vertex, edges, W, eps):
  Xp = jnp.pad(X, ((0, R_PAD - N), (0, 0)))
  npad = NNZ_PAD - vertex.shape[0]
  fill = TRASH + (jnp.arange(npad, dtype=jnp.int32) % (R_PAD - N))
  v3 = jnp.concatenate([vertex.astype(jnp.int32), fill]).reshape(NW, K, CHUNK)
  e3 = jnp.concatenate([edges.astype(jnp.int32), fill]).reshape(NW, K, CHUNK)

  idx_ve = jnp.stack([v3, e3], axis=2).reshape(NW, K // 8, 8, 2, CHUNK)
  idx_ev = jnp.stack([e3, v3], axis=2).reshape(NW, K // 8, 8, 2, CHUNK)
  sums, cnts = _make_scatter(True)(Xp, idx_ve)
  Xe = _tc_norm(sums[0], sums[1], cnts.reshape(NW, R_PAD))
  xv = _make_scatter(False)(Xe, idx_ev)
  out = _matmul(eps, Xp, xv[0], xv[1], W)
  return out[:N]


# speedup vs baseline: 4.0481x; 1.0112x over previous
import functools

import jax
import jax.numpy as jnp
from jax import lax
from jax.experimental import pallas as pl
from jax.experimental.pallas import tpu as pltpu
from jax.experimental.pallas import tpu_sc as plsc

NC = 2
NS = 16
L = 16
NW = NC * NS

N = 10000
E = 10000
D = 128
R_PAD = 10240
TRASH = 10000
CHUNK = 128
K = 80
NNZ_PAD = NW * K * CHUNK
RPT = R_PAD // NS
RPW = R_PAD // NW
NB = 64


def _mesh():
  return plsc.VectorSubcoreMesh(
      core_axis_name="c", subcore_axis_name="s", num_cores=NC, num_subcores=NS
  )


def _scatter_body(with_counts, *refs):
  if with_counts:
    (src_hbm, idx_hbm, out_sums, out_cnt,
     idx_v, rows2_v, cnt_v,
     acc_sh, sem_g0, sem_g1) = refs
  else:
    (src_hbm, idx_hbm, out_sums,
     idx_v, rows2_v,
     acc_sh, sem_g0, sem_g1) = refs
    cnt_v = None
  rows_v = rows2_v.at[0]

  c = lax.axis_index("c")
  s = lax.axis_index("s")
  wid = c * NS + s
  base = s * RPT

  zv = jnp.zeros((L,), jnp.float32)

  def zero_row(i, carry):
    for j in range(D // L):
      rows_v[i, pl.ds(j * L, L)] = zv
    return carry

  lax.fori_loop(0, CHUNK, zero_row, 0)

  if with_counts:
    def zero_cnt(i, carry):
      cnt_v[pl.ds(i * L, L)] = zv
      return carry

    lax.fori_loop(0, R_PAD // L, zero_cnt, 0)

  for k in range(RPT // CHUNK):
    pltpu.sync_copy(rows_v, acc_sh.at[pl.ds(base + k * CHUNK, CHUNK)])

  pltpu.sync_copy(idx_hbm.at[wid, 0], idx_v.at[0])

  plsc.subcore_barrier()

  ones = jnp.ones((L,), jnp.float32)

  pltpu.async_copy(src_hbm.at[idx_v.at[0, 0, 0]], rows2_v.at[0], sem_g0)
  pltpu.async_copy(src_hbm.at[idx_v.at[0, 1, 0]], rows2_v.at[1], sem_g1)

  def do_counts(blk, row):
    for t in range(CHUNK // L):
      idx = idx_v[blk, row, 1, pl.ds(t * L, L)]
      plsc.addupdate_scatter(cnt_v, [idx], ones)

  def half(j, buf, sem):
    blk = jnp.bitwise_and(lax.shift_right_logical(j, 3), 1)
    row = jnp.bitwise_and(j, 7)
    pltpu.make_async_copy(src_hbm.at[idx_v.at[blk, row, 0]], buf, sem).wait()
    pltpu.sync_copy(buf, acc_sh.at[idx_v.at[blk, row, 1]], add=True)

    @pl.when(j + 2 < K)
    def _():
      blk2 = jnp.bitwise_and(lax.shift_right_logical(j + 2, 3), 1)
      row2 = jnp.bitwise_and(j + 2, 7)
      pltpu.async_copy(src_hbm.at[idx_v.at[blk2, row2, 0]], buf, sem)

    if with_counts:
      do_counts(blk, row)

  def pair(i, carry):
    j = 2 * i

    @pl.when(jnp.logical_and(jnp.bitwise_and(j, 7) == 6, j + 2 < K))
    def _():
      nxt = lax.shift_right_logical(j + 2, 3)
      pltpu.sync_copy(idx_hbm.at[wid, nxt],
                      idx_v.at[jnp.bitwise_and(nxt, 1)])

    half(j, rows2_v.at[0], sem_g0)
    half(j + 1, rows2_v.at[1], sem_g1)
    return carry

  lax.fori_loop(0, K // 2, pair, 0)

  if with_counts:
    pltpu.sync_copy(cnt_v, out_cnt.at[pl.ds(wid * R_PAD, R_PAD)])

  plsc.subcore_barrier()

  for k in range(RPT // CHUNK):
    r0 = base + k * CHUNK
    pltpu.sync_copy(acc_sh.at[pl.ds(r0, CHUNK)], rows_v)
    pltpu.sync_copy(rows_v, out_sums.at[c, pl.ds(r0, CHUNK)])


def _make_scatter(with_counts):
  outs = [jax.ShapeDtypeStruct((NC, R_PAD, D), jnp.float32)]
  scratch = [
      pltpu.VMEM((2, 8, 2, CHUNK), jnp.int32),
      pltpu.VMEM((2, CHUNK, D), jnp.float32),
  ]
  if with_counts:
    outs.append(jax.ShapeDtypeStruct((NW * R_PAD,), jnp.float32))
    scratch.append(pltpu.VMEM((R_PAD,), jnp.float32))
  scratch.append(pltpu.VMEM_SHARED((R_PAD, D), jnp.float32))
  scratch += [pltpu.SemaphoreType.DMA] * 2
  return pl.kernel(
      functools.partial(_scatter_body, with_counts),
      out_type=tuple(outs) if with_counts else outs[0],
      mesh=_mesh(),
      scratch_types=scratch,
      compiler_params=pltpu.CompilerParams(needs_layout_passes=False),
  )


def _tc_norm_body(s0_ref, s1_ref, cnt_ref, o_ref):
  c = jnp.sum(cnt_ref[...], axis=0)
  scale = 1.0 / jnp.maximum(c, 1.0)
  o_ref[...] = (s0_ref[...] + s1_ref[...]) * scale[:, None]


def _tc_norm(s0, s1, cnt2):
  BR = 2048
  return pl.pallas_call(
      _tc_norm_body,
      grid=(R_PAD // BR,),
      in_specs=[
          pl.BlockSpec((BR, D), lambda i: (i, 0)),
          pl.BlockSpec((BR, D), lambda i: (i, 0)),
          pl.BlockSpec((NW, BR), lambda i: (0, i)),
      ],
      out_specs=pl.BlockSpec((BR, D), lambda i: (i, 0)),
      out_shape=jax.ShapeDtypeStruct((R_PAD, D), jnp.float32),
  )(s0, s1, cnt2)


def _mm_body(eps_ref, x_ref, v0_ref, v1_ref, w_ref, o_ref):
  scale = 1.0 + eps_ref[0]
  acc = scale * x_ref[...] + v0_ref[...] + v1_ref[...]
  o_ref[...] = lax.dot_general(
      acc, w_ref[...], (((1,), (1,)), ((), ())),
      preferred_element_type=jnp.float32,
  )


def _matmul(eps, Xp, v0, v1, W):
  M = Xp.shape[0]
  BM = 2048
  return pl.pallas_call(
      _mm_body,
      grid=(M // BM,),
      in_specs=[
          pl.BlockSpec(memory_space=pltpu.SMEM),
          pl.BlockSpec((BM, D), lambda i: (i, 0)),
          pl.BlockSpec((BM, D), lambda i: (i, 0)),
          pl.BlockSpec((BM, D), lambda i: (i, 0)),
          pl.BlockSpec((D, D), lambda i: (0, 0)),
      ],
      out_specs=pl.BlockSpec((BM, D), lambda i: (i, 0)),
      out_shape=jax.ShapeDtypeStruct((M, D), jnp.float32),
  )(eps, Xp, v0, v1, W)


def kernel(X, vertex, edges, W, eps):
  Xp = jnp.pad(X, ((0, R_PAD - N), (0, 0)))
  npad = NNZ_PAD - vertex.shape[0]
  fill = TRASH + (jnp.arange(npad, dtype=jnp.int32) % (R_PAD - N))
  v3 = jnp.concatenate([vertex.astype(jnp.int32), fill]).reshape(NW, K, CHUNK)
  e3 = jnp.concatenate([edges.astype(jnp.int32), fill]).reshape(NW, K, CHUNK)

  idx_ve = jnp.stack([v3, e3], axis=2).reshape(NW, K // 8, 8, 2, CHUNK)
  idx_ev = jnp.stack([e3, v3], axis=2).reshape(NW, K // 8, 8, 2, CHUNK)
  sums, cnts = _make_scatter(True)(Xp, idx_ve)
  Xe = _tc_norm(sums[0], sums[1], cnts.reshape(NW, R_PAD))
  xv = _make_scatter(False)(Xe, idx_ev)
  out = _matmul(eps, Xp, xv[0], xv[1], W)
  return out[:N]
